# baseline (speedup 1.0000x reference)
import jax
import jax.numpy as jnp
from jax.experimental import pallas as pl

_BT = 512
_CP = 8


def _body(x_ref, w_ref, o_ref):
    o_ref[...] = jax.lax.dot_general(
        x_ref[...], w_ref[...],
        dimension_numbers=(((1,), (1,)), ((), ())),
        preferred_element_type=jnp.float32,
    )


def kernel(x, weight):
    B, D = x.shape
    C = weight.shape[0]
    w_pad = jnp.zeros((_CP, D), dtype=x.dtype).at[:C, :].set(weight)
    out = pl.pallas_call(
        _body,
        grid=(B // _BT,),
        in_specs=[
            pl.BlockSpec((_BT, D), lambda i: (i, 0)),
            pl.BlockSpec((_CP, D), lambda i: (0, 0)),
        ],
        out_specs=pl.BlockSpec((_BT, _CP), lambda i: (i, 0)),
        out_shape=jax.ShapeDtypeStruct((B, _CP), jnp.float32),
    )(x, w_pad)
    return out[:, :C]


# speedup vs baseline: 1.0051x; 1.0051x over previous
import jax
import jax.numpy as jnp
from jax.experimental import pallas as pl

_BT = 512
_CP = 8


def _body(x_ref, w_ref, o_ref):
    o_ref[...] = jax.lax.dot_general(
        x_ref[...].astype(jnp.bfloat16), w_ref[...].astype(jnp.bfloat16),
        dimension_numbers=(((1,), (1,)), ((), ())),
        preferred_element_type=jnp.float32,
    )


def kernel(x, weight):
    B, D = x.shape
    C = weight.shape[0]
    w_pad = jnp.zeros((_CP, D), dtype=x.dtype).at[:C, :].set(weight)
    out = pl.pallas_call(
        _body,
        grid=(B // _BT,),
        in_specs=[
            pl.BlockSpec((_BT, D), lambda i: (i, 0)),
            pl.BlockSpec((_CP, D), lambda i: (0, 0)),
        ],
        out_specs=pl.BlockSpec((_BT, _CP), lambda i: (i, 0)),
        out_shape=jax.ShapeDtypeStruct((B, _CP), jnp.float32),
    )(x, w_pad)
    return out[:, :C]
